# baseline (speedup 1.0000x reference)
import functools

import jax
import jax.numpy as jnp
from jax import lax
from jax.experimental import pallas as pl
from jax.experimental.pallas import tpu as pltpu
from jax.experimental.pallas import tpu_sc as plsc

E = 1_600_000
PER = 32
D = 3 * PER
NC, NS = 2, 16
NW = NC * NS
B_PER_W = E // NW
CHUNK = 400
N_CHUNK = B_PER_W // CHUNK


def _build_fused_table(aromatic_table, conjugated_table, stereo_table):
    def body(a_ref, c_ref, s_ref, o_ref):
        k = lax.broadcasted_iota(jnp.int32, (64, 1), 0)
        ia = k // 21
        ic = (k // 7) % 3
        iz = k % 7
        a_emb = jnp.zeros((64, PER), jnp.float32)
        c_emb = jnp.zeros((64, PER), jnp.float32)
        s_emb = jnp.zeros((64, PER), jnp.float32)
        for r in range(3):
            a_emb = a_emb + jnp.where(ia == r, 1.0, 0.0) * a_ref[r, :][None, :]
            c_emb = c_emb + jnp.where(ic == r, 1.0, 0.0) * c_ref[r, :][None, :]
        for r in range(7):
            s_emb = s_emb + jnp.where(iz == r, 1.0, 0.0) * s_ref[r, :][None, :]
        pad = jnp.zeros((64, 128 - D), jnp.float32)
        o_ref[...] = jnp.concatenate([a_emb, c_emb, s_emb, pad], axis=1)

    return pl.pallas_call(
        body,
        out_shape=jax.ShapeDtypeStruct((64, 128), jnp.float32),
    )(aromatic_table, conjugated_table, stereo_table)


_SC_MESH = plsc.VectorSubcoreMesh(core_axis_name="c", subcore_axis_name="s")


@functools.partial(
    pl.kernel,
    out_type=jax.ShapeDtypeStruct((D, E), jnp.float32),
    mesh=_SC_MESH,
    scratch_types=[
        pltpu.VMEM((CHUNK,), jnp.int32), pltpu.VMEM((CHUNK,), jnp.int32),
        pltpu.VMEM((CHUNK,), jnp.int32), pltpu.VMEM((CHUNK,), jnp.int32),
        pltpu.VMEM((CHUNK,), jnp.int32), pltpu.VMEM((CHUNK,), jnp.int32),
        pltpu.VMEM((D, CHUNK), jnp.float32), pltpu.VMEM((D, CHUNK), jnp.float32),
        pltpu.VMEM((64, 128), jnp.float32),
        pltpu.VMEM((64 * D,), jnp.float32),
        pltpu.SemaphoreType.DMA, pltpu.SemaphoreType.DMA,
        pltpu.SemaphoreType.DMA, pltpu.SemaphoreType.DMA,
    ],
    compiler_params=pltpu.CompilerParams(use_tc_tiling_on_sc=False,
                                         needs_layout_passes=False),
)
def _sc_embed(a_hbm, c_hbm, s_hbm, fused_hbm, out_hbm,
              a0, c0, s0, a1, c1, s1, t0, t1, fused2d, fused_flat,
              si0, si1, so0, so1):
    wid = lax.axis_index("s") * NC + lax.axis_index("c")
    base_w = wid * B_PER_W
    bufs = ((a0, c0, s0, t0, si0, so0),
            (a1, c1, s1, t1, si1, so1))

    pltpu.sync_copy(fused_hbm, fused2d)
    for r in range(64):
        for cc in range(0, D, 16):
            fused_flat[pl.ds(r * D + cc, 16)] = fused2d[r, pl.ds(cc, 16)]

    def start_idx(ii, b):
        a_v, c_v, s_v, _, si, _ = bufs[b]
        base = base_w + ii * CHUNK
        pltpu.async_copy(a_hbm.at[pl.ds(base, CHUNK)], a_v, si)
        pltpu.async_copy(c_hbm.at[pl.ds(base, CHUNK)], c_v, si)
        pltpu.async_copy(s_hbm.at[pl.ds(base, CHUNK)], s_v, si)

    def wait_idx(ii, b):
        a_v, c_v, s_v, _, si, _ = bufs[b]
        base = base_w + ii * CHUNK
        pltpu.make_async_copy(a_hbm.at[pl.ds(base, CHUNK)], a_v, si).wait()
        pltpu.make_async_copy(c_hbm.at[pl.ds(base, CHUNK)], c_v, si).wait()
        pltpu.make_async_copy(s_hbm.at[pl.ds(base, CHUNK)], s_v, si).wait()

    def wait_out(ii, b):
        colsT, so = bufs[b][3], bufs[b][5]
        base = base_w + ii * CHUNK
        pltpu.make_async_copy(colsT, out_hbm.at[:, pl.ds(base, CHUNK)],
                              so).wait()

    def process(ii, b, first_round):
        a_v, c_v, s_v, colsT, si, so = bufs[b]
        base = base_w + ii * CHUNK
        wait_idx(ii, b)
        if not first_round:
            wait_out(ii - 2, b)

        @pl.loop(0, CHUNK // 16)
        def _tile(t):
            sl = pl.ds(t * 16, 16)
            comb = a_v[sl] * 21 + c_v[sl] * 7 + s_v[sl]
            vaddr = comb * D
            for j in range(D):
                colsT[j, sl] = plsc.load_gather(fused_flat, [vaddr])
                if j < D - 1:
                    vaddr = vaddr + 1

        pltpu.async_copy(colsT, out_hbm.at[:, pl.ds(base, CHUNK)], so)

    start_idx(0, 0)
    start_idx(1, 1)
    process(0, 0, True)
    start_idx(2, 0)
    process(1, 1, True)
    start_idx(3, 1)

    @pl.loop(1, (N_CHUNK - 1) // 2)
    def _pair(p):
        for b in (0, 1):
            ii = 2 * p + b
            process(ii, b, False)

            @pl.when(ii + 2 < N_CHUNK)
            def _prefetch():
                start_idx(ii + 2, b)

    process(N_CHUNK - 1, 0, False)
    wait_out(N_CHUNK - 2, 1)
    wait_out(N_CHUNK - 1, 0)


def kernel(prop_bond_aromatic, prop_bond_conjugated, prop_bond_stereo,
           aromatic_table, conjugated_table, stereo_table):
    fused = _build_fused_table(aromatic_table, conjugated_table, stereo_table)
    out_t = _sc_embed(
        prop_bond_aromatic.astype(jnp.int32),
        prop_bond_conjugated.astype(jnp.int32),
        prop_bond_stereo.astype(jnp.int32),
        fused,
    )
    return out_t.T


# speedup vs baseline: 8.4035x; 8.4035x over previous
import functools

import jax
import jax.numpy as jnp
from jax import lax
from jax.experimental import pallas as pl
from jax.experimental.pallas import tpu as pltpu
from jax.experimental.pallas import tpu_sc as plsc

E = 1_600_000
PER = 32
D = 3 * PER
NC, NS = 2, 16
NW = NC * NS
B_PER_W = E // NW
CHUNK = 400
GSUB = 80
N_CHUNK = B_PER_W // CHUNK


def _build_fused_table(aromatic_table, conjugated_table, stereo_table):
    def body(a_ref, c_ref, s_ref, o_ref):
        k = lax.broadcasted_iota(jnp.int32, (64, 1), 0)
        ia = k // 21
        ic = (k // 7) % 3
        iz = k % 7
        a_emb = jnp.zeros((64, PER), jnp.float32)
        c_emb = jnp.zeros((64, PER), jnp.float32)
        s_emb = jnp.zeros((64, PER), jnp.float32)
        for r in range(3):
            a_emb = a_emb + jnp.where(ia == r, 1.0, 0.0) * a_ref[r, :][None, :]
            c_emb = c_emb + jnp.where(ic == r, 1.0, 0.0) * c_ref[r, :][None, :]
        for r in range(7):
            s_emb = s_emb + jnp.where(iz == r, 1.0, 0.0) * s_ref[r, :][None, :]
        pad = jnp.zeros((64, 128 - D), jnp.float32)
        o_ref[...] = jnp.concatenate([a_emb, c_emb, s_emb, pad], axis=1)

    return pl.pallas_call(
        body,
        out_shape=jax.ShapeDtypeStruct((64, 128), jnp.float32),
    )(aromatic_table, conjugated_table, stereo_table)


_SC_MESH = plsc.VectorSubcoreMesh(core_axis_name="c", subcore_axis_name="s")


@functools.partial(
    pl.kernel,
    out_type=jax.ShapeDtypeStruct((E, 128), jnp.float32),
    mesh=_SC_MESH,
    scratch_types=[
        pltpu.VMEM((CHUNK,), jnp.int32), pltpu.VMEM((CHUNK,), jnp.int32),
        pltpu.VMEM((CHUNK,), jnp.int32), pltpu.VMEM((CHUNK,), jnp.int32),
        pltpu.VMEM((CHUNK,), jnp.int32), pltpu.VMEM((CHUNK,), jnp.int32),
        pltpu.VMEM((CHUNK,), jnp.int32), pltpu.VMEM((CHUNK,), jnp.int32),
        pltpu.VMEM((CHUNK, 128), jnp.float32), pltpu.VMEM((CHUNK, 128), jnp.float32),
        pltpu.VMEM_SHARED((64, 128), jnp.float32),
        pltpu.SemaphoreType.DMA, pltpu.SemaphoreType.DMA,
        pltpu.SemaphoreType.DMA, pltpu.SemaphoreType.DMA,
        pltpu.SemaphoreType.DMA, pltpu.SemaphoreType.DMA,
    ],
)
def _sc_embed(a_hbm, c_hbm, s_hbm, fused_hbm, out_hbm,
              a0, c0, s0, k0, a1, c1, s1, k1, r0, r1, fused_v,
              si0, si1, sg0, sg1, so0, so1):
    wid = lax.axis_index("s") * NC + lax.axis_index("c")
    base_w = wid * B_PER_W
    bufs = ((a0, c0, s0, k0, r0, si0, sg0, so0),
            (a1, c1, s1, k1, r1, si1, sg1, so1))

    @pl.when(lax.axis_index("s") == 0)
    def _stage_table():
        pltpu.sync_copy(fused_hbm, fused_v)

    plsc.subcore_barrier()

    def start_idx(ii, b):
        a_v, c_v, s_v = bufs[b][0], bufs[b][1], bufs[b][2]
        si = bufs[b][5]
        base = base_w + ii * CHUNK
        pltpu.async_copy(a_hbm.at[pl.ds(base, CHUNK)], a_v, si)
        pltpu.async_copy(c_hbm.at[pl.ds(base, CHUNK)], c_v, si)
        pltpu.async_copy(s_hbm.at[pl.ds(base, CHUNK)], s_v, si)

    def wait_idx(ii, b):
        a_v, c_v, s_v = bufs[b][0], bufs[b][1], bufs[b][2]
        si = bufs[b][5]
        base = base_w + ii * CHUNK
        pltpu.make_async_copy(a_hbm.at[pl.ds(base, CHUNK)], a_v, si).wait()
        pltpu.make_async_copy(c_hbm.at[pl.ds(base, CHUNK)], c_v, si).wait()
        pltpu.make_async_copy(s_hbm.at[pl.ds(base, CHUNK)], s_v, si).wait()

    def wait_out(ii, b):
        rows_v, so = bufs[b][4], bufs[b][7]
        base = base_w + ii * CHUNK
        pltpu.make_async_copy(rows_v, out_hbm.at[pl.ds(base, CHUNK)], so).wait()

    def process(ii, b, first_round):
        a_v, c_v, s_v, comb_v, rows_v, si, sg, so = bufs[b]
        base = base_w + ii * CHUNK
        wait_idx(ii, b)

        @pl.loop(0, CHUNK, step=16)
        def _combine(j):
            sl = pl.ds(j, 16)
            comb_v[sl] = a_v[sl] * 21 + c_v[sl] * 7 + s_v[sl]

        if not first_round:
            wait_out(ii - 2, b)
        gathers = [
            pltpu.async_copy(
                fused_v.at[comb_v.at[pl.ds(g * GSUB, GSUB)]],
                rows_v.at[pl.ds(g * GSUB, GSUB)],
                sg,
            )
            for g in range(CHUNK // GSUB)
        ]
        for cp in gathers:
            cp.wait()
        pltpu.async_copy(rows_v, out_hbm.at[pl.ds(base, CHUNK)], so)

    start_idx(0, 0)
    start_idx(1, 1)
    process(0, 0, True)
    start_idx(2, 0)
    process(1, 1, True)
    start_idx(3, 1)

    @pl.loop(1, (N_CHUNK - 1) // 2)
    def _pair(p):
        for b in (0, 1):
            ii = 2 * p + b
            process(ii, b, False)

            @pl.when(ii + 2 < N_CHUNK)
            def _prefetch():
                start_idx(ii + 2, b)

    process(N_CHUNK - 1, 0, False)
    wait_out(N_CHUNK - 2, 1)
    wait_out(N_CHUNK - 1, 0)


def _tc_compact_transpose(out_pad):
    BLK = 512

    def body(x_ref, o_ref):
        x = x_ref[...]
        o_ref[...] = x.T[:D, :]

    return pl.pallas_call(
        body,
        grid=(E // BLK,),
        in_specs=[pl.BlockSpec((BLK, 128), lambda i: (i, 0))],
        out_specs=pl.BlockSpec((D, BLK), lambda i: (0, i)),
        out_shape=jax.ShapeDtypeStruct((D, E), jnp.float32),
    )(out_pad)


def kernel(prop_bond_aromatic, prop_bond_conjugated, prop_bond_stereo,
           aromatic_table, conjugated_table, stereo_table):
    fused = _build_fused_table(aromatic_table, conjugated_table, stereo_table)
    out_pad = _sc_embed(
        prop_bond_aromatic.astype(jnp.int32),
        prop_bond_conjugated.astype(jnp.int32),
        prop_bond_stereo.astype(jnp.int32),
        fused,
    )
    return _tc_compact_transpose(out_pad).T


# speedup vs baseline: 16.8293x; 2.0027x over previous
import functools

import jax
import jax.numpy as jnp
from jax import lax
from jax.experimental import pallas as pl
from jax.experimental.pallas import tpu as pltpu
from jax.experimental.pallas import tpu_sc as plsc

E = 1_600_000
PER = 32
D = 3 * PER
NC, NS = 2, 16
NW = NC * NS
B_PER_W = E // NW
CHUNK = 400
GSUB = 80
N_CHUNK = B_PER_W // CHUNK


def _build_fused_table(aromatic_table, conjugated_table, stereo_table):
    def body(a_ref, c_ref, s_ref, o_ref):
        k = lax.broadcasted_iota(jnp.int32, (64, 1), 0)
        ia = k // 21
        ic = (k // 7) % 3
        iz = k % 7
        a_emb = jnp.zeros((64, PER), jnp.float32)
        c_emb = jnp.zeros((64, PER), jnp.float32)
        s_emb = jnp.zeros((64, PER), jnp.float32)
        for r in range(3):
            a_emb = a_emb + jnp.where(ia == r, 1.0, 0.0) * a_ref[r, :][None, :]
            c_emb = c_emb + jnp.where(ic == r, 1.0, 0.0) * c_ref[r, :][None, :]
        for r in range(7):
            s_emb = s_emb + jnp.where(iz == r, 1.0, 0.0) * s_ref[r, :][None, :]
        pad = jnp.zeros((64, 128 - D), jnp.float32)
        o_ref[...] = jnp.concatenate([a_emb, c_emb, s_emb, pad], axis=1)

    return pl.pallas_call(
        body,
        out_shape=jax.ShapeDtypeStruct((64, 128), jnp.float32),
    )(aromatic_table, conjugated_table, stereo_table)


_SC_MESH = plsc.VectorSubcoreMesh(core_axis_name="c", subcore_axis_name="s")


@functools.partial(
    pl.kernel,
    out_type=jax.ShapeDtypeStruct((E, 128), jnp.float32),
    mesh=_SC_MESH,
    scratch_types=[
        pltpu.VMEM((CHUNK,), jnp.int32), pltpu.VMEM((CHUNK,), jnp.int32),
        pltpu.VMEM((CHUNK,), jnp.int32), pltpu.VMEM((CHUNK,), jnp.int32),
        pltpu.VMEM((CHUNK,), jnp.int32), pltpu.VMEM((CHUNK,), jnp.int32),
        pltpu.VMEM((CHUNK,), jnp.int32), pltpu.VMEM((CHUNK,), jnp.int32),
        pltpu.VMEM((CHUNK, 128), jnp.float32), pltpu.VMEM((CHUNK, 128), jnp.float32),
        pltpu.VMEM_SHARED((64, 128), jnp.float32),
        pltpu.SemaphoreType.DMA, pltpu.SemaphoreType.DMA,
        pltpu.SemaphoreType.DMA, pltpu.SemaphoreType.DMA,
        pltpu.SemaphoreType.DMA, pltpu.SemaphoreType.DMA,
    ],
)
def _sc_embed(a_hbm, c_hbm, s_hbm, fused_hbm, out_hbm,
              a0, c0, s0, k0, a1, c1, s1, k1, r0, r1, fused_v,
              si0, si1, sg0, sg1, so0, so1):
    wid = lax.axis_index("s") * NC + lax.axis_index("c")
    base_w = wid * B_PER_W
    bufs = ((a0, c0, s0, k0, r0, si0, sg0, so0),
            (a1, c1, s1, k1, r1, si1, sg1, so1))

    @pl.when(lax.axis_index("s") == 0)
    def _stage_table():
        pltpu.sync_copy(fused_hbm, fused_v)

    plsc.subcore_barrier()

    def start_idx(ii, b):
        a_v, c_v, s_v = bufs[b][0], bufs[b][1], bufs[b][2]
        si = bufs[b][5]
        base = base_w + ii * CHUNK
        pltpu.async_copy(a_hbm.at[pl.ds(base, CHUNK)], a_v, si)
        pltpu.async_copy(c_hbm.at[pl.ds(base, CHUNK)], c_v, si)
        pltpu.async_copy(s_hbm.at[pl.ds(base, CHUNK)], s_v, si)

    def wait_idx(ii, b):
        a_v, c_v, s_v = bufs[b][0], bufs[b][1], bufs[b][2]
        si = bufs[b][5]
        base = base_w + ii * CHUNK
        pltpu.make_async_copy(a_hbm.at[pl.ds(base, CHUNK)], a_v, si).wait()
        pltpu.make_async_copy(c_hbm.at[pl.ds(base, CHUNK)], c_v, si).wait()
        pltpu.make_async_copy(s_hbm.at[pl.ds(base, CHUNK)], s_v, si).wait()

    def wait_out(ii, b):
        rows_v, so = bufs[b][4], bufs[b][7]
        base = base_w + ii * CHUNK
        pltpu.make_async_copy(rows_v, out_hbm.at[pl.ds(base, CHUNK)], so).wait()

    def process(ii, b, first_round):
        a_v, c_v, s_v, comb_v, rows_v, si, sg, so = bufs[b]
        base = base_w + ii * CHUNK
        wait_idx(ii, b)

        @pl.loop(0, CHUNK, step=16)
        def _combine(j):
            sl = pl.ds(j, 16)
            comb_v[sl] = a_v[sl] * 21 + c_v[sl] * 7 + s_v[sl]

        if not first_round:
            wait_out(ii - 2, b)
        gathers = [
            pltpu.async_copy(
                fused_v.at[comb_v.at[pl.ds(g * GSUB, GSUB)]],
                rows_v.at[pl.ds(g * GSUB, GSUB)],
                sg,
            )
            for g in range(CHUNK // GSUB)
        ]
        for cp in gathers:
            cp.wait()
        pltpu.async_copy(rows_v, out_hbm.at[pl.ds(base, CHUNK)], so)

    start_idx(0, 0)
    start_idx(1, 1)
    process(0, 0, True)
    start_idx(2, 0)
    process(1, 1, True)
    start_idx(3, 1)

    @pl.loop(1, (N_CHUNK - 1) // 2)
    def _pair(p):
        for b in (0, 1):
            ii = 2 * p + b
            process(ii, b, False)

            @pl.when(ii + 2 < N_CHUNK)
            def _prefetch():
                start_idx(ii + 2, b)

    process(N_CHUNK - 1, 0, False)
    wait_out(N_CHUNK - 2, 1)
    wait_out(N_CHUNK - 1, 0)


def _tc_compact_transpose(out_pad):
    BLK = 2048

    def body(x_ref, o_ref):
        o_ref[...] = x_ref[...][:, :D].T

    return pl.pallas_call(
        body,
        grid=(E // BLK,),
        in_specs=[pl.BlockSpec((BLK, 128), lambda i: (i, 0))],
        out_specs=pl.BlockSpec((D, BLK), lambda i: (0, i)),
        out_shape=jax.ShapeDtypeStruct((D, E), jnp.float32),
    )(out_pad)


def kernel(prop_bond_aromatic, prop_bond_conjugated, prop_bond_stereo,
           aromatic_table, conjugated_table, stereo_table):
    fused = _build_fused_table(aromatic_table, conjugated_table, stereo_table)
    out_pad = _sc_embed(
        prop_bond_aromatic.astype(jnp.int32),
        prop_bond_conjugated.astype(jnp.int32),
        prop_bond_stereo.astype(jnp.int32),
        fused,
    )
    return _tc_compact_transpose(out_pad).T


# speedup vs baseline: 23.0765x; 1.3712x over previous
import functools

import jax
import jax.numpy as jnp
from jax import lax
from jax.experimental import pallas as pl
from jax.experimental.pallas import tpu as pltpu
from jax.experimental.pallas import tpu_sc as plsc

E = 1_600_000
PER = 32
D = 3 * PER
NC, NS = 2, 16
NW = NC * NS
B_PER_W = E // NW
CHUNK = 400
GSUB = 80
N_CHUNK = B_PER_W // CHUNK


def _build_fused_table(aromatic_table, conjugated_table, stereo_table):
    def body(a_ref, c_ref, s_ref, o_ref):
        k = lax.broadcasted_iota(jnp.int32, (64, 1), 0)
        ia = k // 21
        ic = (k // 7) % 3
        iz = k % 7
        a_emb = jnp.zeros((64, PER), jnp.float32)
        c_emb = jnp.zeros((64, PER), jnp.float32)
        s_emb = jnp.zeros((64, PER), jnp.float32)
        for r in range(3):
            a_emb = a_emb + jnp.where(ia == r, 1.0, 0.0) * a_ref[r, :][None, :]
            c_emb = c_emb + jnp.where(ic == r, 1.0, 0.0) * c_ref[r, :][None, :]
        for r in range(7):
            s_emb = s_emb + jnp.where(iz == r, 1.0, 0.0) * s_ref[r, :][None, :]
        pad = jnp.zeros((64, 128 - D), jnp.float32)
        o_ref[...] = jnp.concatenate([a_emb, c_emb, s_emb, pad], axis=1)

    return pl.pallas_call(
        body,
        out_shape=jax.ShapeDtypeStruct((64, 128), jnp.float32),
    )(aromatic_table, conjugated_table, stereo_table)


_SC_MESH = plsc.VectorSubcoreMesh(core_axis_name="c", subcore_axis_name="s")


@functools.partial(
    pl.kernel,
    out_type=jax.ShapeDtypeStruct((E, 128), jnp.float32),
    mesh=_SC_MESH,
    scratch_types=[
        pltpu.VMEM((CHUNK,), jnp.int32), pltpu.VMEM((CHUNK,), jnp.int32),
        pltpu.VMEM((CHUNK,), jnp.int32), pltpu.VMEM((CHUNK,), jnp.int32),
        pltpu.VMEM((CHUNK,), jnp.int32), pltpu.VMEM((CHUNK,), jnp.int32),
        pltpu.VMEM((CHUNK,), jnp.int32), pltpu.VMEM((CHUNK,), jnp.int32),
        pltpu.VMEM((CHUNK, 128), jnp.float32), pltpu.VMEM((CHUNK, 128), jnp.float32),
        pltpu.VMEM_SHARED((64, 128), jnp.float32),
        pltpu.SemaphoreType.DMA, pltpu.SemaphoreType.DMA,
        pltpu.SemaphoreType.DMA, pltpu.SemaphoreType.DMA,
        pltpu.SemaphoreType.DMA, pltpu.SemaphoreType.DMA,
    ],
)
def _sc_embed(a_hbm, c_hbm, s_hbm, fused_hbm, out_hbm,
              a0, c0, s0, k0, a1, c1, s1, k1, r0, r1, fused_v,
              si0, si1, sg0, sg1, so0, so1):
    wid = lax.axis_index("s") * NC + lax.axis_index("c")
    base_w = wid * B_PER_W
    bufs = ((a0, c0, s0, k0, r0, si0, sg0, so0),
            (a1, c1, s1, k1, r1, si1, sg1, so1))

    @pl.when(lax.axis_index("s") == 0)
    def _stage_table():
        pltpu.sync_copy(fused_hbm, fused_v)

    plsc.subcore_barrier()

    def start_idx(ii, b):
        a_v, c_v, s_v = bufs[b][0], bufs[b][1], bufs[b][2]
        si = bufs[b][5]
        base = base_w + ii * CHUNK
        pltpu.async_copy(a_hbm.at[pl.ds(base, CHUNK)], a_v, si)
        pltpu.async_copy(c_hbm.at[pl.ds(base, CHUNK)], c_v, si)
        pltpu.async_copy(s_hbm.at[pl.ds(base, CHUNK)], s_v, si)

    def wait_idx(ii, b):
        a_v, c_v, s_v = bufs[b][0], bufs[b][1], bufs[b][2]
        si = bufs[b][5]
        base = base_w + ii * CHUNK
        pltpu.make_async_copy(a_hbm.at[pl.ds(base, CHUNK)], a_v, si).wait()
        pltpu.make_async_copy(c_hbm.at[pl.ds(base, CHUNK)], c_v, si).wait()
        pltpu.make_async_copy(s_hbm.at[pl.ds(base, CHUNK)], s_v, si).wait()

    def wait_out(ii, b):
        rows_v, so = bufs[b][4], bufs[b][7]
        base = base_w + ii * CHUNK
        pltpu.make_async_copy(rows_v, out_hbm.at[pl.ds(base, CHUNK)], so).wait()

    def process(ii, b, first_round):
        a_v, c_v, s_v, comb_v, rows_v, si, sg, so = bufs[b]
        base = base_w + ii * CHUNK
        wait_idx(ii, b)

        @pl.loop(0, CHUNK, step=16)
        def _combine(j):
            sl = pl.ds(j, 16)
            comb_v[sl] = a_v[sl] * 21 + c_v[sl] * 7 + s_v[sl]

        if not first_round:
            wait_out(ii - 2, b)
        gathers = [
            pltpu.async_copy(
                fused_v.at[comb_v.at[pl.ds(g * GSUB, GSUB)]],
                rows_v.at[pl.ds(g * GSUB, GSUB)],
                sg,
            )
            for g in range(CHUNK // GSUB)
        ]
        for cp in gathers:
            cp.wait()
        pltpu.async_copy(rows_v, out_hbm.at[pl.ds(base, CHUNK)], so)

    start_idx(0, 0)
    start_idx(1, 1)
    process(0, 0, True)
    start_idx(2, 0)
    process(1, 1, True)
    start_idx(3, 1)

    @pl.loop(1, (N_CHUNK - 1) // 2)
    def _pair(p):
        for b in (0, 1):
            ii = 2 * p + b
            process(ii, b, False)

            @pl.when(ii + 2 < N_CHUNK)
            def _prefetch():
                start_idx(ii + 2, b)

    process(N_CHUNK - 1, 0, False)
    wait_out(N_CHUNK - 2, 1)
    wait_out(N_CHUNK - 1, 0)


def kernel(prop_bond_aromatic, prop_bond_conjugated, prop_bond_stereo,
           aromatic_table, conjugated_table, stereo_table):
    fused = _build_fused_table(aromatic_table, conjugated_table, stereo_table)
    out_pad = _sc_embed(
        prop_bond_aromatic.astype(jnp.int32),
        prop_bond_conjugated.astype(jnp.int32),
        prop_bond_stereo.astype(jnp.int32),
        fused,
    )
    return out_pad[:, :D]
